# Initial kernel scaffold; baseline (speedup 1.0000x reference)
import jax
import jax.numpy as jnp
from jax.experimental import pallas as pl


def kernel(energy, n_atoms, Z, image_idx, scale_by, shift_by, atomic_energies):
    raise NotImplementedError("write your pallas kernel here")


# speedup vs baseline: 13.7389x; 13.7389x over previous
import functools

import jax
import jax.numpy as jnp
from jax import lax
from jax.experimental import pallas as pl
from jax.experimental.pallas import tpu as pltpu
from jax.experimental.pallas import tpu_sc as plsc

NG = 4096
NE_PAD = 128
NC, NS, L = 2, 16, 16
NW = NC * NS
CH = 3200
K = CH // 128
ATOT_PAD = NW * CH
ROW = 4104
GPW = NG // NW

@functools.cache
def _build_pass1():
  mesh = plsc.VectorSubcoreMesh(
      core_axis_name="c", subcore_axis_name="s",
      num_cores=NC, num_subcores=NS)

  @functools.partial(
      pl.kernel,
      out_type=jax.ShapeDtypeStruct((NW, ROW), jnp.float32),
      mesh=mesh,
      compiler_params=pltpu.CompilerParams(needs_layout_passes=False),
      scratch_types=[
          pltpu.VMEM((CH,), jnp.int32),
          pltpu.VMEM((CH,), jnp.int32),
          pltpu.VMEM((NE_PAD,), jnp.float32),
          pltpu.VMEM((ROW,), jnp.float32),
      ],
  )
  def _pass1(z_hbm, img_hbm, ae_hbm, zrow_hbm, out_hbm,
             z_v, g_v, ae_v, acc_v):
    c = lax.axis_index("c")
    s = lax.axis_index("s")
    w = s * NC + c
    pltpu.sync_copy(z_hbm.at[pl.ds(w * CH, CH)], z_v)
    pltpu.sync_copy(img_hbm.at[pl.ds(w * CH, CH)], g_v)
    pltpu.sync_copy(ae_hbm, ae_v)
    pltpu.sync_copy(zrow_hbm, acc_v)
    for i in range(CH // L):
        sl = pl.ds(i * L, L)
        vals = plsc.load_gather(ae_v, [z_v[sl]])
        plsc.addupdate_scatter(acc_v, [g_v[sl]], vals)
    pltpu.sync_copy(acc_v, out_hbm.at[w])

  return _pass1


@functools.cache
def _build_pass2():
  mesh = plsc.VectorSubcoreMesh(
      core_axis_name="c", subcore_axis_name="s",
      num_cores=NC, num_subcores=NS)

  @functools.partial(
      pl.kernel,
      out_type=jax.ShapeDtypeStruct((NG,), jnp.float32),
      mesh=mesh,
      compiler_params=pltpu.CompilerParams(needs_layout_passes=False),
      scratch_types=[
          pltpu.VMEM((NW, GPW), jnp.float32),
          pltpu.VMEM((GPW,), jnp.float32),
          pltpu.VMEM((GPW,), jnp.int32),
          pltpu.VMEM((L,), jnp.float32),
          pltpu.VMEM((L,), jnp.float32),
          pltpu.VMEM((GPW,), jnp.float32),
      ],
  )
  def _pass2(part_hbm, energy_hbm, natoms_hbm, scale_hbm, shift_hbm, out_hbm,
             blk_v, en_v, na_v, sc_v, sh_v, res_v):
    c = lax.axis_index("c")
    s = lax.axis_index("s")
    w = s * NC + c
    g0 = w * GPW
    for r in range(NW):
        pltpu.sync_copy(part_hbm.at[r, pl.ds(g0, GPW)], blk_v.at[r])
    pltpu.sync_copy(energy_hbm.at[pl.ds(g0, GPW)], en_v)
    pltpu.sync_copy(natoms_hbm.at[pl.ds(g0, GPW)], na_v)
    pltpu.sync_copy(scale_hbm, sc_v)
    pltpu.sync_copy(shift_hbm, sh_v)
    scale = sc_v[pl.ds(0, L)]
    shift = sh_v[pl.ds(0, L)]
    for k in range(GPW // L):
        sl = pl.ds(k * L, L)
        acc = blk_v[0, sl]
        for r in range(1, NW):
            acc = acc + blk_v[r, sl]
        res_v[sl] = en_v[sl] * scale + na_v[sl].astype(jnp.float32) * shift + acc
    pltpu.sync_copy(res_v, out_hbm.at[pl.ds(g0, GPW)])

  return _pass2


def kernel(energy, n_atoms, Z, image_idx, scale_by, shift_by, atomic_energies):
    n_atoms_total = Z.shape[0]
    pad = ATOT_PAD - n_atoms_total
    z32 = jnp.concatenate(
        [Z.astype(jnp.int32), jnp.zeros((pad,), jnp.int32)])
    img32 = jnp.concatenate(
        [image_idx.astype(jnp.int32), jnp.full((pad,), NG, jnp.int32)])
    ae = jnp.zeros((NE_PAD,), jnp.float32).at[: atomic_energies.shape[0]].set(
        atomic_energies)
    zrow = jnp.zeros((ROW,), jnp.float32)
    part = _build_pass1()(z32, img32, ae, zrow)
    scale = jnp.broadcast_to(scale_by.astype(jnp.float32), (L,))
    shift = jnp.broadcast_to(shift_by.astype(jnp.float32), (L,))
    return _build_pass2()(part, energy, n_atoms, scale, shift)
